# baseline (speedup 1.0000x reference)
import functools

import jax
import jax.numpy as jnp
from jax import lax
from jax.experimental import pallas as pl
from jax.experimental.pallas import tpu as pltpu
from jax.experimental.pallas import tpu_sc as plsc

B = 4096
S = 200
E = 32
L = 16

NC = 2
NS = 16
NW = NC * NS

BBLK = B // NW
E0 = E // 8
CS = 4
NCHUNK = S // CS
ROWS = CS * BBLK
PB = BBLK + 1
TPAD = CS * E0 * 8 * PB


def _fire_gathers(tok_hbm, idx_v, slab_v, gsem):
    for si in range(CS):
        pltpu.make_async_copy(
            tok_hbm.at[idx_v.at[si]],
            slab_v.at[pl.ds(si * BBLK, BBLK)],
            gsem,
        ).start()


def _drain(hbm_dummy, vmem_ref, sem):
    pltpu.make_async_copy(hbm_dummy, vmem_ref, sem).wait()


def _transpose_add(slab_v, tpad_v, tbuf_v, pos_v, s0):
    iot = lax.iota(jnp.int32, L)
    coffs = []
    for h in range(2):
        ev = iot + (h * L)
        coffs.append((ev >> 3) * (8 * PB) + (ev & 7) * PB)

    def b_body(b1, c):
        for si in range(CS):
            base = si * (E0 * 8 * PB) + b1
            row = si * BBLK + b1
            for h in range(2):
                pv = plsc.load_gather(
                    pos_v, [iot + (h * L), jnp.broadcast_to(s0 + si, (L,))])
                v = slab_v[row, pl.ds(h * L, L)] + pv
                plsc.store_scatter(tpad_v, [coffs[h] + base], v)
        return c

    lax.fori_loop(0, BBLK, b_body, 0)

    def c_body(r, c):
        src = r * PB
        si = r // (E0 * 8)
        eh = (r // 8) % E0
        el = r % 8
        for bg in range(BBLK // L):
            tbuf_v[si, eh, el, pl.ds(bg * L, L)] = (
                tpad_v[pl.ds(src + bg * L, L)])
        return c

    lax.fori_loop(0, CS * E0 * 8, c_body, 0)


def _sc_body(seq_hbm, tok_hbm, pos_hbm, out_hbm,
             idx0, idx1, slab0, slab1, tpad_v, tbuf_v, pos_v,
             gsem0, gsem1, osem):
    wid = lax.axis_index("s") * NC + lax.axis_index("c")
    idxs = (idx0, idx1)
    slabs = (slab0, slab1)
    gsems = (gsem0, gsem1)

    pltpu.sync_copy(pos_hbm, pos_v)

    def stage_and_fire(g, buf):
        s0 = g * CS
        pltpu.sync_copy(
            seq_hbm.at[pl.ds(s0, CS), pl.ds(wid * BBLK, BBLK)], idxs[buf])
        _fire_gathers(tok_hbm, idxs[buf], slabs[buf], gsems[buf])

    def process(g, buf):
        s0 = g * CS
        _drain(tok_hbm.at[pl.ds(0, ROWS)], slabs[buf], gsems[buf])

        @pl.when(g + 1 < NCHUNK)
        def _():
            stage_and_fire(g + 1, 1 - buf)

        @pl.when(g >= 1)
        def _():
            _drain(out_hbm.at[pl.ds(0, CS), :, 0], tbuf_v, osem)

        _transpose_add(slabs[buf], tpad_v, tbuf_v, pos_v, s0)

        pltpu.make_async_copy(
            tbuf_v, out_hbm.at[pl.ds(s0, CS), :, wid], osem).start()

    stage_and_fire(0, 0)

    def outer(gg, carry):
        process(gg * 2, 0)
        process(gg * 2 + 1, 1)
        return carry

    lax.fori_loop(0, NCHUNK // 2, outer, 0)

    _drain(out_hbm.at[pl.ds(0, CS), :, 0], tbuf_v, osem)


@jax.jit
def _sc_embed(seqT, token_table, posT):
    mesh = plsc.VectorSubcoreMesh(
        core_axis_name="c", subcore_axis_name="s", num_cores=NC, num_subcores=NS
    )
    return pl.kernel(
        _sc_body,
        out_type=jax.ShapeDtypeStruct((S, E0, NW, 8, BBLK), jnp.float32),
        mesh=mesh,
        compiler_params=pltpu.CompilerParams(
            use_tc_tiling_on_sc=False, needs_layout_passes=False),
        scratch_types=[
            pltpu.VMEM((CS, BBLK), jnp.int32),
            pltpu.VMEM((CS, BBLK), jnp.int32),
            pltpu.VMEM((ROWS, E), jnp.float32),
            pltpu.VMEM((ROWS, E), jnp.float32),
            pltpu.VMEM((TPAD,), jnp.float32),
            pltpu.VMEM((CS, E0, 8, BBLK), jnp.float32),
            pltpu.VMEM((E, S), jnp.float32),
            pltpu.SemaphoreType.DMA,
            pltpu.SemaphoreType.DMA,
            pltpu.SemaphoreType.DMA,
        ],
    )(seqT, token_table, posT)


def kernel(seq, token_table, pos_table):
    out5 = _sc_embed(jnp.transpose(seq), token_table, jnp.transpose(pos_table))
    return out5.transpose(2, 4, 0, 1, 3).reshape(B, S, E)


# speedup vs baseline: 1.1192x; 1.1192x over previous
import functools

import jax
import jax.numpy as jnp
from jax import lax
from jax.experimental import pallas as pl
from jax.experimental.pallas import tpu as pltpu
from jax.experimental.pallas import tpu_sc as plsc

B = 4096
S = 200
E = 32
L = 16

NC = 2
NS = 16
NW = NC * NS

BBLK = B // NW
E0 = E // 8
CS = 4
NCHUNK = S // CS
ROWS = CS * BBLK
PB = BBLK + 1
TPAD = CS * E0 * 8 * PB


def _fire_gathers(tok_hbm, idx_v, slab_v, gsem):
    for si in range(CS):
        pltpu.make_async_copy(
            tok_hbm.at[idx_v.at[si]],
            slab_v.at[pl.ds(si * BBLK, BBLK)],
            gsem,
        ).start()


def _drain(hbm_dummy, vmem_ref, sem):
    pltpu.make_async_copy(hbm_dummy, vmem_ref, sem).wait()


def _transpose_add(slab_v, tpad_v, tbuf_v, pos_v, s0):
    iot = lax.iota(jnp.int32, L)
    coffs = []
    for h in range(2):
        ev = iot + (h * L)
        coffs.append((ev >> 3) * (8 * PB) + (ev & 7) * PB)

    pvs = tuple(
        plsc.load_gather(
            pos_v, [iot + (h * L), jnp.broadcast_to(s0 + si, (L,))])
        for si in range(CS) for h in range(2))

    def b_body(b1, pcarry):
        for si in range(CS):
            base = si * (E0 * 8 * PB) + b1
            row = si * BBLK + b1
            for h in range(2):
                v = slab_v[row, pl.ds(h * L, L)] + pcarry[si * 2 + h]
                plsc.store_scatter(tpad_v, [coffs[h] + base], v)
        return pcarry

    lax.fori_loop(0, BBLK, b_body, pvs)

    def c_body(r, c):
        src = r * PB
        si = r // (E0 * 8)
        eh = (r // 8) % E0
        el = r % 8
        for bg in range(BBLK // L):
            tbuf_v[si, eh, el, pl.ds(bg * L, L)] = (
                tpad_v[pl.ds(src + bg * L, L)])
        return c

    lax.fori_loop(0, CS * E0 * 8, c_body, 0)


def _sc_body(seq_hbm, tok_hbm, pos_hbm, out_hbm,
             idx0, idx1, slab0, slab1, tpad_v, tbuf_v, pos_v,
             gsem0, gsem1, osem):
    wid = lax.axis_index("s") * NC + lax.axis_index("c")
    idxs = (idx0, idx1)
    slabs = (slab0, slab1)
    gsems = (gsem0, gsem1)

    pltpu.sync_copy(pos_hbm, pos_v)

    def stage_and_fire(g, buf):
        s0 = g * CS
        pltpu.sync_copy(
            seq_hbm.at[pl.ds(s0, CS), pl.ds(wid * BBLK, BBLK)], idxs[buf])
        _fire_gathers(tok_hbm, idxs[buf], slabs[buf], gsems[buf])

    def process(g, buf):
        s0 = g * CS
        _drain(tok_hbm.at[pl.ds(0, ROWS)], slabs[buf], gsems[buf])

        @pl.when(g + 1 < NCHUNK)
        def _():
            stage_and_fire(g + 1, 1 - buf)

        @pl.when(g >= 1)
        def _():
            _drain(out_hbm.at[pl.ds(0, CS), :, 0], tbuf_v, osem)

        _transpose_add(slabs[buf], tpad_v, tbuf_v, pos_v, s0)

        pltpu.make_async_copy(
            tbuf_v, out_hbm.at[pl.ds(s0, CS), :, wid], osem).start()

    stage_and_fire(0, 0)

    def outer(gg, carry):
        process(gg * 2, 0)
        process(gg * 2 + 1, 1)
        return carry

    lax.fori_loop(0, NCHUNK // 2, outer, 0)

    _drain(out_hbm.at[pl.ds(0, CS), :, 0], tbuf_v, osem)


@jax.jit
def _sc_embed(seqT, token_table, posT):
    mesh = plsc.VectorSubcoreMesh(
        core_axis_name="c", subcore_axis_name="s", num_cores=NC, num_subcores=NS
    )
    return pl.kernel(
        _sc_body,
        out_type=jax.ShapeDtypeStruct((S, E0, NW, 8, BBLK), jnp.float32),
        mesh=mesh,
        compiler_params=pltpu.CompilerParams(
            use_tc_tiling_on_sc=False, needs_layout_passes=False),
        scratch_types=[
            pltpu.VMEM((CS, BBLK), jnp.int32),
            pltpu.VMEM((CS, BBLK), jnp.int32),
            pltpu.VMEM((ROWS, E), jnp.float32),
            pltpu.VMEM((ROWS, E), jnp.float32),
            pltpu.VMEM((TPAD,), jnp.float32),
            pltpu.VMEM((CS, E0, 8, BBLK), jnp.float32),
            pltpu.VMEM((E, S), jnp.float32),
            pltpu.SemaphoreType.DMA,
            pltpu.SemaphoreType.DMA,
            pltpu.SemaphoreType.DMA,
        ],
    )(seqT, token_table, posT)


def kernel(seq, token_table, pos_table):
    out5 = _sc_embed(jnp.transpose(seq), token_table, jnp.transpose(pos_table))
    return out5.transpose(2, 4, 0, 1, 3).reshape(B, S, E)


# speedup vs baseline: 1.1270x; 1.0070x over previous
import functools

import jax
import jax.numpy as jnp
from jax import lax
from jax.experimental import pallas as pl
from jax.experimental.pallas import tpu as pltpu
from jax.experimental.pallas import tpu_sc as plsc

B = 4096
S = 200
E = 32
L = 16

NC = 2
NS = 16
NW = NC * NS

BBLK = B // NW
E0 = E // 8
CS = 4
NCHUNK = S // CS
ROWS = CS * BBLK
PB = BBLK + 1
TPAD = CS * E0 * 8 * PB


def _fire_gathers(tok_hbm, idx_v, slab_v, gsem):
    for si in range(CS):
        pltpu.make_async_copy(
            tok_hbm.at[idx_v.at[si]],
            slab_v.at[pl.ds(si * BBLK, BBLK)],
            gsem,
        ).start()


def _drain(hbm_dummy, vmem_ref, sem):
    pltpu.make_async_copy(hbm_dummy, vmem_ref, sem).wait()


def _transpose_add(slab_v, tpad_v, tbuf_v, pos_v, s0):
    iot = lax.iota(jnp.int32, L)
    coffs = []
    for h in range(2):
        ev = iot + (h * L)
        coffs.append((ev >> 3) * (8 * PB) + (ev & 7) * PB)

    pvs = tuple(
        plsc.load_gather(
            pos_v, [iot + (h * L), jnp.broadcast_to(s0 + si, (L,))])
        for si in range(CS) for h in range(2))

    def b_body(b1, pcarry):
        for si in range(CS):
            base = si * (E0 * 8 * PB) + b1
            row = si * BBLK + b1
            for h in range(2):
                v = slab_v[row, pl.ds(h * L, L)] + pcarry[si * 2 + h]
                plsc.store_scatter(tpad_v, [coffs[h] + base], v)
        return pcarry

    lax.fori_loop(0, BBLK, b_body, pvs, unroll=4)

    def c_body(r, c):
        src = r * PB
        si = r // (E0 * 8)
        eh = (r // 8) % E0
        el = r % 8
        for bg in range(BBLK // L):
            tbuf_v[si, eh, el, pl.ds(bg * L, L)] = (
                tpad_v[pl.ds(src + bg * L, L)])
        return c

    lax.fori_loop(0, CS * E0 * 8, c_body, 0, unroll=4)


def _sc_body(seq_hbm, tok_hbm, pos_hbm, out_hbm,
             idx0, idx1, slab0, slab1, tpad_v, tbuf_v, pos_v,
             gsem0, gsem1, osem):
    wid = lax.axis_index("s") * NC + lax.axis_index("c")
    idxs = (idx0, idx1)
    slabs = (slab0, slab1)
    gsems = (gsem0, gsem1)

    pltpu.sync_copy(pos_hbm, pos_v)

    def stage_and_fire(g, buf):
        s0 = g * CS
        pltpu.sync_copy(
            seq_hbm.at[pl.ds(s0, CS), pl.ds(wid * BBLK, BBLK)], idxs[buf])
        _fire_gathers(tok_hbm, idxs[buf], slabs[buf], gsems[buf])

    def process(g, buf):
        s0 = g * CS
        _drain(tok_hbm.at[pl.ds(0, ROWS)], slabs[buf], gsems[buf])

        @pl.when(g + 1 < NCHUNK)
        def _():
            stage_and_fire(g + 1, 1 - buf)

        @pl.when(g >= 1)
        def _():
            _drain(out_hbm.at[pl.ds(0, CS), :, 0], tbuf_v, osem)

        _transpose_add(slabs[buf], tpad_v, tbuf_v, pos_v, s0)

        pltpu.make_async_copy(
            tbuf_v, out_hbm.at[pl.ds(s0, CS), :, wid], osem).start()

    stage_and_fire(0, 0)

    def outer(gg, carry):
        process(gg * 2, 0)
        process(gg * 2 + 1, 1)
        return carry

    lax.fori_loop(0, NCHUNK // 2, outer, 0)

    _drain(out_hbm.at[pl.ds(0, CS), :, 0], tbuf_v, osem)


@jax.jit
def _sc_embed(seqT, token_table, posT):
    mesh = plsc.VectorSubcoreMesh(
        core_axis_name="c", subcore_axis_name="s", num_cores=NC, num_subcores=NS
    )
    return pl.kernel(
        _sc_body,
        out_type=jax.ShapeDtypeStruct((S, E0, NW, 8, BBLK), jnp.float32),
        mesh=mesh,
        compiler_params=pltpu.CompilerParams(
            use_tc_tiling_on_sc=False, needs_layout_passes=False),
        scratch_types=[
            pltpu.VMEM((CS, BBLK), jnp.int32),
            pltpu.VMEM((CS, BBLK), jnp.int32),
            pltpu.VMEM((ROWS, E), jnp.float32),
            pltpu.VMEM((ROWS, E), jnp.float32),
            pltpu.VMEM((TPAD,), jnp.float32),
            pltpu.VMEM((CS, E0, 8, BBLK), jnp.float32),
            pltpu.VMEM((E, S), jnp.float32),
            pltpu.SemaphoreType.DMA,
            pltpu.SemaphoreType.DMA,
            pltpu.SemaphoreType.DMA,
        ],
    )(seqT, token_table, posT)


def kernel(seq, token_table, pos_table):
    out5 = _sc_embed(jnp.transpose(seq), token_table,
                     jnp.transpose(pos_table))
    return out5.transpose(2, 4, 0, 1, 3).reshape(B, S, E)


# speedup vs baseline: 1.1293x; 1.0020x over previous
import functools

import jax
import jax.numpy as jnp
from jax import lax
from jax.experimental import pallas as pl
from jax.experimental.pallas import tpu as pltpu
from jax.experimental.pallas import tpu_sc as plsc

B = 4096
S = 200
E = 32
L = 16

NC = 2
NS = 16
NW = NC * NS

BBLK = B // NW
E0 = E // 8
CS = 4
NCHUNK = S // CS
ROWS = CS * BBLK
PB = BBLK + 1
TPAD = CS * E0 * 8 * PB


def _fire_gathers(tok_hbm, idx_v, slab_v, gsem):
    for si in range(CS):
        pltpu.make_async_copy(
            tok_hbm.at[idx_v.at[si]],
            slab_v.at[pl.ds(si * BBLK, BBLK)],
            gsem,
        ).start()


def _drain(hbm_dummy, vmem_ref, sem):
    pltpu.make_async_copy(hbm_dummy, vmem_ref, sem).wait()


def _transpose_add(slab_v, tpad_v, tbuf_v, pos_v, s0):
    iot = lax.iota(jnp.int32, L)
    coffs = []
    for h in range(2):
        ev = iot + (h * L)
        coffs.append((ev >> 3) * (8 * PB) + (ev & 7) * PB)

    pvs = tuple(
        plsc.load_gather(
            pos_v, [iot + (h * L), jnp.broadcast_to(s0 + si, (L,))])
        for si in range(CS) for h in range(2))

    def b_body(b1, pcarry):
        for si in range(CS):
            base = si * (E0 * 8 * PB) + b1
            row = si * BBLK + b1
            for h in range(2):
                v = slab_v[row, pl.ds(h * L, L)] + pcarry[si * 2 + h]
                plsc.store_scatter(tpad_v, [coffs[h] + base], v)
        return pcarry

    lax.fori_loop(0, BBLK, b_body, pvs, unroll=8)

    def c_body(r, c):
        src = r * PB
        si = r // (E0 * 8)
        eh = (r // 8) % E0
        el = r % 8
        for bg in range(BBLK // L):
            tbuf_v[si, eh, el, pl.ds(bg * L, L)] = (
                tpad_v[pl.ds(src + bg * L, L)])
        return c

    lax.fori_loop(0, CS * E0 * 8, c_body, 0, unroll=8)


def _sc_body(seq_hbm, tok_hbm, pos_hbm, out_hbm,
             idx0, idx1, slab0, slab1, tpad_v, tbuf_v, pos_v,
             gsem0, gsem1, osem):
    wid = lax.axis_index("s") * NC + lax.axis_index("c")
    idxs = (idx0, idx1)
    slabs = (slab0, slab1)
    gsems = (gsem0, gsem1)

    pltpu.sync_copy(pos_hbm, pos_v)

    def stage_and_fire(g, buf):
        s0 = g * CS
        pltpu.sync_copy(
            seq_hbm.at[pl.ds(s0, CS), pl.ds(wid * BBLK, BBLK)], idxs[buf])
        _fire_gathers(tok_hbm, idxs[buf], slabs[buf], gsems[buf])

    def process(g, buf):
        s0 = g * CS
        _drain(tok_hbm.at[pl.ds(0, ROWS)], slabs[buf], gsems[buf])

        @pl.when(g + 1 < NCHUNK)
        def _():
            stage_and_fire(g + 1, 1 - buf)

        @pl.when(g >= 1)
        def _():
            _drain(out_hbm.at[pl.ds(0, CS), :, 0], tbuf_v, osem)

        _transpose_add(slabs[buf], tpad_v, tbuf_v, pos_v, s0)

        pltpu.make_async_copy(
            tbuf_v, out_hbm.at[pl.ds(s0, CS), :, wid], osem).start()

    stage_and_fire(0, 0)

    def outer(gg, carry):
        process(gg * 2, 0)
        process(gg * 2 + 1, 1)
        return carry

    lax.fori_loop(0, NCHUNK // 2, outer, 0)

    _drain(out_hbm.at[pl.ds(0, CS), :, 0], tbuf_v, osem)


@jax.jit
def _sc_embed(seqT, token_table, posT):
    mesh = plsc.VectorSubcoreMesh(
        core_axis_name="c", subcore_axis_name="s", num_cores=NC, num_subcores=NS
    )
    return pl.kernel(
        _sc_body,
        out_type=jax.ShapeDtypeStruct((S, E0, NW, 8, BBLK), jnp.float32),
        mesh=mesh,
        compiler_params=pltpu.CompilerParams(
            use_tc_tiling_on_sc=False, needs_layout_passes=False),
        scratch_types=[
            pltpu.VMEM((CS, BBLK), jnp.int32),
            pltpu.VMEM((CS, BBLK), jnp.int32),
            pltpu.VMEM((ROWS, E), jnp.float32),
            pltpu.VMEM((ROWS, E), jnp.float32),
            pltpu.VMEM((TPAD,), jnp.float32),
            pltpu.VMEM((CS, E0, 8, BBLK), jnp.float32),
            pltpu.VMEM((E, S), jnp.float32),
            pltpu.SemaphoreType.DMA,
            pltpu.SemaphoreType.DMA,
            pltpu.SemaphoreType.DMA,
        ],
    )(seqT, token_table, posT)


def kernel(seq, token_table, pos_table):
    out5 = _sc_embed(jnp.transpose(seq), token_table,
                     jnp.transpose(pos_table))
    return out5.transpose(2, 4, 0, 1, 3).reshape(B, S, E)


# speedup vs baseline: 1.6406x; 1.4529x over previous
import functools

import jax
import jax.numpy as jnp
from jax import lax
from jax.experimental import pallas as pl
from jax.experimental.pallas import tpu as pltpu
from jax.experimental.pallas import tpu_sc as plsc

B = 4096
S = 200
E = 32
L = 16

NC = 2
NS = 16
NW = NC * NS

BBLK = B // NW
E0 = E // 8
CS = 4
NCHUNK = S // CS
ROWS = CS * BBLK
PB = BBLK + 1
TPAD = CS * E0 * 8 * PB


def _fire_gathers(tok_hbm, idx_v, slab_v, gsem):
    for si in range(CS):
        pltpu.make_async_copy(
            tok_hbm.at[idx_v.at[si]],
            slab_v.at[pl.ds(si * BBLK, BBLK)],
            gsem,
        ).start()


def _drain(hbm_dummy, vmem_ref, sem):
    pltpu.make_async_copy(hbm_dummy, vmem_ref, sem).wait()


def _transpose_add(slab_v, tpad_v, tbuf_v, pos_v, s0):
    iot = lax.iota(jnp.int32, L)
    coffs = []
    for h in range(2):
        ev = iot + (h * L)
        coffs.append((ev >> 3) * (8 * PB) + (ev & 7) * PB)

    pvs = tuple(
        plsc.load_gather(
            pos_v, [iot + (h * L), jnp.broadcast_to(s0 + si, (L,))])
        for si in range(CS) for h in range(2))

    def b_body(b1, pcarry):
        vs = [slab_v[si * BBLK + b1, pl.ds(h * L, L)]
              for si in range(CS) for h in range(2)]
        for si in range(CS):
            base = si * (E0 * 8 * PB) + b1
            for h in range(2):
                plsc.store_scatter(tpad_v, [coffs[h] + base],
                                   vs[si * 2 + h] + pcarry[si * 2 + h])
        return pcarry

    lax.fori_loop(0, BBLK, b_body, pvs, unroll=8)

    def c_body(r, c):
        src = r * PB
        si = r // (E0 * 8)
        eh = (r // 8) % E0
        el = r % 8
        vs = [tpad_v[pl.ds(src + bg * L, L)] for bg in range(BBLK // L)]
        for bg in range(BBLK // L):
            tbuf_v[si, eh, el, pl.ds(bg * L, L)] = vs[bg]
        return c

    lax.fori_loop(0, CS * E0 * 8, c_body, 0, unroll=8)


def _sc_body(seq_hbm, tok_hbm, pos_hbm, out_hbm,
             idx0, idx1, slab0, slab1, tpad_v, tbuf_v, pos_v,
             gsem0, gsem1, osem):
    wid = lax.axis_index("s") * NC + lax.axis_index("c")
    idxs = (idx0, idx1)
    slabs = (slab0, slab1)
    gsems = (gsem0, gsem1)

    pltpu.sync_copy(pos_hbm, pos_v)

    def stage_and_fire(g, buf):
        s0 = g * CS
        pltpu.sync_copy(
            seq_hbm.at[pl.ds(s0, CS), pl.ds(wid * BBLK, BBLK)], idxs[buf])
        _fire_gathers(tok_hbm, idxs[buf], slabs[buf], gsems[buf])

    def process(g, buf):
        s0 = g * CS
        _drain(tok_hbm.at[pl.ds(0, ROWS)], slabs[buf], gsems[buf])

        @pl.when(g + 1 < NCHUNK)
        def _():
            stage_and_fire(g + 1, 1 - buf)

        @pl.when(g >= 1)
        def _():
            _drain(out_hbm.at[pl.ds(0, CS), :, 0], tbuf_v, osem)

        _transpose_add(slabs[buf], tpad_v, tbuf_v, pos_v, s0)

        pltpu.make_async_copy(
            tbuf_v, out_hbm.at[pl.ds(s0, CS), :, wid], osem).start()

    stage_and_fire(0, 0)

    def outer(gg, carry):
        process(gg * 2, 0)
        process(gg * 2 + 1, 1)
        return carry

    lax.fori_loop(0, NCHUNK // 2, outer, 0)

    _drain(out_hbm.at[pl.ds(0, CS), :, 0], tbuf_v, osem)


@jax.jit
def _sc_embed(seqT, token_table, posT):
    mesh = plsc.VectorSubcoreMesh(
        core_axis_name="c", subcore_axis_name="s", num_cores=NC, num_subcores=NS
    )
    return pl.kernel(
        _sc_body,
        out_type=jax.ShapeDtypeStruct((S, E0, NW, 8, BBLK), jnp.float32),
        mesh=mesh,
        compiler_params=pltpu.CompilerParams(
            use_tc_tiling_on_sc=False, needs_layout_passes=False),
        scratch_types=[
            pltpu.VMEM((CS, BBLK), jnp.int32),
            pltpu.VMEM((CS, BBLK), jnp.int32),
            pltpu.VMEM((ROWS, E), jnp.float32),
            pltpu.VMEM((ROWS, E), jnp.float32),
            pltpu.VMEM((TPAD,), jnp.float32),
            pltpu.VMEM((CS, E0, 8, BBLK), jnp.float32),
            pltpu.VMEM((E, S), jnp.float32),
            pltpu.SemaphoreType.DMA,
            pltpu.SemaphoreType.DMA,
            pltpu.SemaphoreType.DMA,
        ],
    )(seqT, token_table, posT)


def kernel(seq, token_table, pos_table):
    out5 = _sc_embed(jnp.transpose(seq), token_table,
                     jnp.transpose(pos_table))
    return out5.transpose(2, 4, 0, 1, 3).reshape(B, S, E)
